# Initial kernel scaffold; baseline (speedup 1.0000x reference)
import jax
import jax.numpy as jnp
from jax.experimental import pallas as pl


def kernel(type_indices, adjacency, table, W_msg, b_msg, W_upd, b_upd, gamma, beta):
    raise NotImplementedError("write your pallas kernel here")


# speedup vs baseline: 2.4669x; 2.4669x over previous
import functools

import jax
import jax.numpy as jnp
from jax import lax
from jax.experimental import pallas as pl
from jax.experimental.pallas import tpu as pltpu
from jax.experimental.pallas import tpu_sc as plsc

N = 10000
E = 160000
D = 128
LAYERS = 3
T = 64

NC = 2
NS = 16
NW = NC * NS
VL = 16

CK = 64
ECHUNKS = 80
EW = CK * ECHUNKS
EPAD = EW * NW

R = 10240
RW = R // NS

NCHUNK_INIT = 5
NPAD = NW * NCHUNK_INIT * CK

DBLK = 1000
NBLK = N // DBLK

def _mesh():
    return plsc.VectorSubcoreMesh(core_axis_name="c", subcore_axis_name="s",
                                  num_cores=NC, num_subcores=NS)


@functools.lru_cache(maxsize=None)
def _make_sc_init():
    @functools.partial(
        pl.kernel,
        out_type=(
            jax.ShapeDtypeStruct((NPAD, D), jnp.float32),
            jax.ShapeDtypeStruct((NPAD, D), jnp.float32),
            jax.ShapeDtypeStruct((NPAD, D), jnp.float32),
        ),
        mesh=_mesh(),
        scratch_types=[
            pltpu.VMEM((CK,), jnp.int32),
            pltpu.VMEM((CK, D), jnp.float32),
            pltpu.SemaphoreType.DMA,
        ],
    )
    def _sc_init(tabx, taba, tabb, idx_h, outx, outa, outb, idxv, buf, sem):
        wid = lax.axis_index("s") * NC + lax.axis_index("c")
        for c in range(NCHUNK_INIT):
            base = (wid * NCHUNK_INIT + c) * CK
            pltpu.sync_copy(idx_h.at[pl.ds(base, CK)], idxv)
            for tab, out in ((tabx, outx), (taba, outa), (tabb, outb)):
                pltpu.async_copy(tab.at[idxv], buf, sem).wait()
                pltpu.sync_copy(buf, out.at[pl.ds(base, CK)])

    return _sc_init


@functools.lru_cache(maxsize=None)
def _make_edge_kernel():
    def body(a_hbm, b_hbm, src_h, tgtg_h, tgts_h, bias_hbm, zeros_hbm,
             part, msgacc, srcx, tgtx, scatx, rowsa, rowsb, biasv, sema,
             semb):
        cid = lax.axis_index("c")
        sid = lax.axis_index("s")
        wid = sid * NC + cid

        pltpu.sync_copy(zeros_hbm, rowsa)

        def zloop(k, carry):
            pltpu.sync_copy(rowsa, msgacc.at[pl.ds(sid * RW + k * CK, CK)])
            return carry

        lax.fori_loop(0, RW // CK, zloop, 0)

        pltpu.sync_copy(bias_hbm, biasv)
        plsc.subcore_barrier()

        bjs = [biasv[pl.ds(j * VL, VL)] for j in range(D // VL)]

        def chunk(c, carry):
            off = wid * EW + c * CK
            pltpu.sync_copy(src_h.at[pl.ds(off, CK)], srcx)
            pltpu.sync_copy(tgtg_h.at[pl.ds(off, CK)], tgtx)
            pltpu.sync_copy(tgts_h.at[pl.ds(off, CK)], scatx)
            ca = pltpu.async_copy(a_hbm.at[srcx], rowsa, sema)
            cb = pltpu.async_copy(b_hbm.at[tgtx], rowsb, semb)
            ca.wait()
            cb.wait()

            def rloop(r, rc):
                for j in range(D // VL):
                    sl = pl.ds(j * VL, VL)
                    rowsa[r, sl] = jnp.maximum(
                        rowsa[r, sl] + rowsb[r, sl] + bjs[j], 0.0)
                return rc

            lax.fori_loop(0, CK, rloop, 0)
            pltpu.sync_copy(rowsa, msgacc.at[scatx], add=True)
            return carry

        lax.fori_loop(0, ECHUNKS, chunk, 0)
        plsc.subcore_barrier()

        sl = pl.ds(sid * RW, RW)
        pltpu.sync_copy(msgacc.at[sl], part.at[cid, sl])

    return pl.kernel(
        body,
        out_type=jax.ShapeDtypeStruct((NC, R, D), jnp.float32),
        mesh=_mesh(),
        scratch_types=[
            pltpu.VMEM_SHARED((R, D), jnp.float32),
            pltpu.VMEM((CK,), jnp.int32),
            pltpu.VMEM((CK,), jnp.int32),
            pltpu.VMEM((CK,), jnp.int32),
            pltpu.VMEM((CK, D), jnp.float32),
            pltpu.VMEM((CK, D), jnp.float32),
            pltpu.VMEM((D,), jnp.float32),
            pltpu.SemaphoreType.DMA,
            pltpu.SemaphoreType.DMA,
        ])


@functools.lru_cache(maxsize=None)
def _make_count_kernel():
    def body(tgts_h, ones_hbm, zeros_hbm, cnt, cntacc, scatx, onesv, rowsz):
        cid = lax.axis_index("c")
        sid = lax.axis_index("s")
        wid = sid * NC + cid
        pltpu.sync_copy(zeros_hbm, rowsz)

        def zloop(k, carry):
            pltpu.sync_copy(rowsz, cntacc.at[pl.ds(sid * RW + k * CK, CK)])
            return carry

        lax.fori_loop(0, RW // CK, zloop, 0)
        pltpu.sync_copy(ones_hbm, onesv)
        plsc.subcore_barrier()

        def chunk(c, carry):
            off = wid * EW + c * CK
            pltpu.sync_copy(tgts_h.at[pl.ds(off, CK)], scatx)
            pltpu.sync_copy(onesv, cntacc.at[scatx], add=True)
            return carry

        lax.fori_loop(0, ECHUNKS, chunk, 0)
        plsc.subcore_barrier()
        sl = pl.ds(sid * RW, RW)
        pltpu.sync_copy(cntacc.at[sl], cnt.at[cid, sl])

    return pl.kernel(
        body,
        out_type=jax.ShapeDtypeStruct((NC, R, D), jnp.float32),
        mesh=_mesh(),
        scratch_types=[
            pltpu.VMEM_SHARED((R, D), jnp.float32),
            pltpu.VMEM((CK,), jnp.int32),
            pltpu.VMEM((CK, D), jnp.float32),
            pltpu.VMEM((CK, D), jnp.float32),
        ])


def _make_dense(last, x_rows):
    def body(x_ref, p0, p1, c0, c1, wu1, wu2, bu, gm, bt, *rest):
        if last:
            xn_ref, pool_ref, acc = rest
        else:
            ws, wt = rest[:2]
            xn_ref, a_ref, b_ref = rest[2:]
        x = x_ref[...]
        p = p0[...][0] + p1[...][0]
        c = c0[...][0] + c1[...][0]
        m = p / jnp.maximum(c, 1.0)
        u = jnp.dot(x, wu1[...], preferred_element_type=jnp.float32)
        u += jnp.dot(m, wu2[...], preferred_element_type=jnp.float32)
        u = jnp.maximum(u + bu[...], 0.0)
        h = u + x
        mu = jnp.mean(h, axis=-1, keepdims=True)
        var = jnp.mean((h - mu) ** 2, axis=-1, keepdims=True)
        xn = (h - mu) / jnp.sqrt(var + 1e-5) * gm[...] + bt[...]
        xn_ref[...] = xn
        if last:
            i = pl.program_id(0)
            bs = jnp.sum(xn, axis=0, keepdims=True)
            bm = jnp.max(xn, axis=0, keepdims=True)

            @pl.when(i == 0)
            def _():
                acc[0:1] = bs
                acc[1:2] = bm

            @pl.when(i > 0)
            def _():
                acc[0:1] = acc[0:1] + bs
                acc[1:2] = jnp.maximum(acc[1:2], bm)

            @pl.when(i == NBLK - 1)
            def _():
                pool_ref[0:1] = acc[0:1] * (1.0 / N)
                pool_ref[1:2] = acc[1:2]
        else:
            a_ref[...] = jnp.dot(xn, ws[...],
                                 preferred_element_type=jnp.float32)
            b_ref[...] = jnp.dot(xn, wt[...],
                                 preferred_element_type=jnp.float32)

    full = lambda i: (0, 0)
    in_specs = [
        pl.BlockSpec((DBLK, D), lambda i: (i, 0)),
        pl.BlockSpec((1, DBLK, D), lambda i: (0, i, 0)),
        pl.BlockSpec((1, DBLK, D), lambda i: (1, i, 0)),
        pl.BlockSpec((1, DBLK, D), lambda i: (0, i, 0)),
        pl.BlockSpec((1, DBLK, D), lambda i: (1, i, 0)),
        pl.BlockSpec((D, D), full),
        pl.BlockSpec((D, D), full),
        pl.BlockSpec((1, D), full),
        pl.BlockSpec((1, D), full),
        pl.BlockSpec((1, D), full),
    ]
    out_shape = [jax.ShapeDtypeStruct((N, D), jnp.float32)]
    out_specs = [pl.BlockSpec((DBLK, D), lambda i: (i, 0))]
    scratch_shapes = []
    if last:
        out_shape.append(jax.ShapeDtypeStruct((2, D), jnp.float32))
        out_specs.append(pl.BlockSpec((2, D), full))
        scratch_shapes.append(pltpu.VMEM((2, D), jnp.float32))
    else:
        in_specs += [pl.BlockSpec((D, D), full), pl.BlockSpec((D, D), full)]
        out_shape += [jax.ShapeDtypeStruct((N, D), jnp.float32),
                      jax.ShapeDtypeStruct((N, D), jnp.float32)]
        out_specs += [pl.BlockSpec((DBLK, D), lambda i: (i, 0)),
                      pl.BlockSpec((DBLK, D), lambda i: (i, 0))]
    return pl.pallas_call(
        body,
        grid=(NBLK,),
        in_specs=in_specs,
        out_specs=tuple(out_specs),
        out_shape=tuple(out_shape),
        scratch_shapes=scratch_shapes,
    )


def kernel(type_indices, adjacency, table, W_msg, b_msg, W_upd, b_upd,
           gamma, beta):
    f32 = jnp.float32
    src = adjacency[:, 0]
    tgt = adjacency[:, 1]
    npad = EPAD - E
    zpad = jnp.zeros((npad,), jnp.int32)
    gpad = N + (jnp.arange(npad, dtype=jnp.int32) % (R - N))
    srcp = jnp.concatenate([src, zpad])
    tgtg = jnp.concatenate([tgt, zpad])
    tgts = jnp.concatenate([tgt, gpad])
    ti = jnp.concatenate(
        [type_indices, jnp.zeros((NPAD - N,), jnp.int32)])
    zeros_hbm = jnp.zeros((CK, D), f32)
    ones_hbm = jnp.ones((CK, D), f32)
    WsT = [W_msg[l][:, :D].T for l in range(LAYERS)]
    WtT = [W_msg[l][:, D:].T for l in range(LAYERS)]
    Wu1T = [W_upd[l][:, :D].T for l in range(LAYERS)]
    Wu2T = [W_upd[l][:, D:].T for l in range(LAYERS)]
    tabA0 = table @ WsT[0]
    tabB0 = table @ WtT[0]

    x, A, B = _make_sc_init()(table, tabA0, tabB0, ti)
    cnt = _make_count_kernel()(tgts, ones_hbm, zeros_hbm)
    if isinstance(cnt, (tuple, list)):
        cnt = cnt[0]
    for l in range(LAYERS):
        part = _make_edge_kernel()(A, B, srcp, tgtg, tgts, b_msg[l],
                                   zeros_hbm)
        if isinstance(part, (tuple, list)):
            part = part[0]
        last = l == LAYERS - 1
        dense = _make_dense(last, x.shape[0])
        args = [x, part, part, cnt, cnt, Wu1T[l], Wu2T[l],
                b_upd[l].reshape(1, D), gamma[l].reshape(1, D),
                beta[l].reshape(1, D)]
        if last:
            x, pooled = dense(*args)
        else:
            args += [WsT[l + 1], WtT[l + 1]]
            x, A, B = dense(*args)
    return pooled.reshape(2 * D)


# speedup vs baseline: 3.0797x; 1.2484x over previous
import functools

import jax
import jax.numpy as jnp
from jax import lax
from jax.experimental import pallas as pl
from jax.experimental.pallas import tpu as pltpu
from jax.experimental.pallas import tpu_sc as plsc

N = 10000
E = 160000
D = 128
LAYERS = 3
T = 64

NC = 2
NS = 16
NW = NC * NS
VL = 16

CK = 64
ECHUNKS = 80
EW = CK * ECHUNKS
EPAD = EW * NW

R = 10240
RW = R // NS

NCHUNK_INIT = 5
NPAD = NW * NCHUNK_INIT * CK

DBLK = 1000
NBLK = N // DBLK

def _mesh():
    return plsc.VectorSubcoreMesh(core_axis_name="c", subcore_axis_name="s",
                                  num_cores=NC, num_subcores=NS)


@functools.lru_cache(maxsize=None)
def _make_sc_init():
    @functools.partial(
        pl.kernel,
        out_type=(
            jax.ShapeDtypeStruct((NPAD, D), jnp.float32),
            jax.ShapeDtypeStruct((NPAD, D), jnp.float32),
            jax.ShapeDtypeStruct((NPAD, D), jnp.float32),
        ),
        mesh=_mesh(),
        scratch_types=[
            pltpu.VMEM((CK,), jnp.int32),
            pltpu.VMEM((CK, D), jnp.float32),
            pltpu.SemaphoreType.DMA,
        ],
    )
    def _sc_init(tabx, taba, tabb, idx_h, outx, outa, outb, idxv, buf, sem):
        wid = lax.axis_index("s") * NC + lax.axis_index("c")
        for c in range(NCHUNK_INIT):
            base = (wid * NCHUNK_INIT + c) * CK
            pltpu.sync_copy(idx_h.at[pl.ds(base, CK)], idxv)
            for tab, out in ((tabx, outx), (taba, outa), (tabb, outb)):
                pltpu.async_copy(tab.at[idxv], buf, sem).wait()
                pltpu.sync_copy(buf, out.at[pl.ds(base, CK)])

    return _sc_init


@functools.lru_cache(maxsize=None)
def _make_edge_kernel():
    def body(a_hbm, b_hbm, src_h, tgtg_h, tgts_h, bias_hbm, zeros_hbm,
             part, msgacc, gxs0, gxs1, gxt0, gxt1, scx0, scx1,
             rowsa0, rowsa1, rowsb0, rowsb1, biasv,
             gisem0, gisem1, gsem0, gsem1, ssem0, ssem1):
        cid = lax.axis_index("c")
        sid = lax.axis_index("s")
        wid = sid * NC + cid
        gxs = (gxs0, gxs1)
        gxt = (gxt0, gxt1)
        scx = (scx0, scx1)
        rowsa = (rowsa0, rowsa1)
        rowsb = (rowsb0, rowsb1)
        gisem = (gisem0, gisem1)
        gsem = (gsem0, gsem1)
        ssem = (ssem0, ssem1)

        pltpu.sync_copy(zeros_hbm, rowsa0)

        def zloop(k, carry):
            pltpu.sync_copy(rowsa0, msgacc.at[pl.ds(sid * RW + k * CK, CK)])
            return carry

        lax.fori_loop(0, RW // CK, zloop, 0)

        pltpu.sync_copy(bias_hbm, biasv)
        plsc.subcore_barrier()

        bjs = [biasv[pl.ds(j * VL, VL)] for j in range(D // VL)]

        def choff(cc):
            return wid * EW + jnp.minimum(cc, ECHUNKS - 1) * CK

        def gidx(cc, b):
            off = choff(cc)
            return (pltpu.make_async_copy(src_h.at[pl.ds(off, CK)], gxs[b],
                                          gisem[b]),
                    pltpu.make_async_copy(tgtg_h.at[pl.ds(off, CK)], gxt[b],
                                          gisem[b]))

        def gsc(cc, b):
            return pltpu.make_async_copy(tgts_h.at[pl.ds(choff(cc), CK)],
                                         scx[b], ssem[b])

        def ggat(b):
            return (pltpu.make_async_copy(a_hbm.at[gxs[b]], rowsa[b],
                                          gsem[b]),
                    pltpu.make_async_copy(b_hbm.at[gxt[b]], rowsb[b],
                                          gsem[b]))

        for d in gidx(0, 0) + (gsc(0, 0),):
            d.start()
        for d in gidx(0, 0):
            d.wait()
        for d in ggat(0):
            d.start()
        for d in gidx(1, 1) + (gsc(1, 1),):
            d.start()

        @pl.loop(0, ECHUNKS, step=2)
        def _(c0):
            for b in range(2):
                cc = c0 + b
                nb = 1 - b
                for dd in gidx(cc + 1, nb):
                    dd.wait()
                for dd in ggat(nb):
                    dd.start()
                for dd in ggat(b):
                    dd.wait()
                for dd in gidx(cc + 2, b):
                    dd.start()
                ra, rb = rowsa[b], rowsb[b]

                @pl.loop(0, CK, unroll=4)
                def _(r):
                    for j in range(D // VL):
                        sl = pl.ds(j * VL, VL)
                        ra[r, sl] = jnp.maximum(
                            ra[r, sl] + rb[r, sl] + bjs[j], 0.0)

                gsc(cc, b).wait()
                pltpu.sync_copy(ra, msgacc.at[scx[b]], add=True)
                gsc(cc + 2, b).start()

        for d in ggat(0):
            d.wait()
        for d in gidx(ECHUNKS, 1):
            d.wait()
        gsc(ECHUNKS, 0).wait()
        gsc(ECHUNKS, 1).wait()
        plsc.subcore_barrier()

        sl = pl.ds(sid * RW, RW)
        pltpu.sync_copy(msgacc.at[sl], part.at[cid, sl])

    return pl.kernel(
        body,
        out_type=jax.ShapeDtypeStruct((NC, R, D), jnp.float32),
        mesh=_mesh(),
        scratch_types=[
            pltpu.VMEM_SHARED((R, D), jnp.float32),
            pltpu.VMEM((CK,), jnp.int32),
            pltpu.VMEM((CK,), jnp.int32),
            pltpu.VMEM((CK,), jnp.int32),
            pltpu.VMEM((CK,), jnp.int32),
            pltpu.VMEM((CK,), jnp.int32),
            pltpu.VMEM((CK,), jnp.int32),
            pltpu.VMEM((CK, D), jnp.float32),
            pltpu.VMEM((CK, D), jnp.float32),
            pltpu.VMEM((CK, D), jnp.float32),
            pltpu.VMEM((CK, D), jnp.float32),
            pltpu.VMEM((D,), jnp.float32),
            pltpu.SemaphoreType.DMA,
            pltpu.SemaphoreType.DMA,
            pltpu.SemaphoreType.DMA,
            pltpu.SemaphoreType.DMA,
            pltpu.SemaphoreType.DMA,
            pltpu.SemaphoreType.DMA,
        ])


@functools.lru_cache(maxsize=None)
def _make_count_kernel():
    def body(tgts_h, ones_hbm, zeros_hbm, cnt, cntacc, scatx, onesv, rowsz):
        cid = lax.axis_index("c")
        sid = lax.axis_index("s")
        wid = sid * NC + cid
        pltpu.sync_copy(zeros_hbm, rowsz)

        def zloop(k, carry):
            pltpu.sync_copy(rowsz, cntacc.at[pl.ds(sid * RW + k * CK, CK)])
            return carry

        lax.fori_loop(0, RW // CK, zloop, 0)
        pltpu.sync_copy(ones_hbm, onesv)
        plsc.subcore_barrier()

        def chunk(c, carry):
            off = wid * EW + c * CK
            pltpu.sync_copy(tgts_h.at[pl.ds(off, CK)], scatx)
            pltpu.sync_copy(onesv, cntacc.at[scatx], add=True)
            return carry

        lax.fori_loop(0, ECHUNKS, chunk, 0)
        plsc.subcore_barrier()
        sl = pl.ds(sid * RW, RW)
        pltpu.sync_copy(cntacc.at[sl], cnt.at[cid, sl])

    return pl.kernel(
        body,
        out_type=jax.ShapeDtypeStruct((NC, R, D), jnp.float32),
        mesh=_mesh(),
        scratch_types=[
            pltpu.VMEM_SHARED((R, D), jnp.float32),
            pltpu.VMEM((CK,), jnp.int32),
            pltpu.VMEM((CK, D), jnp.float32),
            pltpu.VMEM((CK, D), jnp.float32),
        ])


def _make_dense(last, x_rows):
    def body(x_ref, p0, p1, c0, c1, wu1, wu2, bu, gm, bt, *rest):
        if last:
            xn_ref, pool_ref, acc = rest
        else:
            ws, wt = rest[:2]
            xn_ref, a_ref, b_ref = rest[2:]
        x = x_ref[...]
        p = p0[...][0] + p1[...][0]
        c = c0[...][0] + c1[...][0]
        m = p / jnp.maximum(c, 1.0)
        u = jnp.dot(x, wu1[...], preferred_element_type=jnp.float32)
        u += jnp.dot(m, wu2[...], preferred_element_type=jnp.float32)
        u = jnp.maximum(u + bu[...], 0.0)
        h = u + x
        mu = jnp.mean(h, axis=-1, keepdims=True)
        var = jnp.mean((h - mu) ** 2, axis=-1, keepdims=True)
        xn = (h - mu) / jnp.sqrt(var + 1e-5) * gm[...] + bt[...]
        xn_ref[...] = xn
        if last:
            i = pl.program_id(0)
            bs = jnp.sum(xn, axis=0, keepdims=True)
            bm = jnp.max(xn, axis=0, keepdims=True)

            @pl.when(i == 0)
            def _():
                acc[0:1] = bs
                acc[1:2] = bm

            @pl.when(i > 0)
            def _():
                acc[0:1] = acc[0:1] + bs
                acc[1:2] = jnp.maximum(acc[1:2], bm)

            @pl.when(i == NBLK - 1)
            def _():
                pool_ref[0:1] = acc[0:1] * (1.0 / N)
                pool_ref[1:2] = acc[1:2]
        else:
            a_ref[...] = jnp.dot(xn, ws[...],
                                 preferred_element_type=jnp.float32)
            b_ref[...] = jnp.dot(xn, wt[...],
                                 preferred_element_type=jnp.float32)

    full = lambda i: (0, 0)
    in_specs = [
        pl.BlockSpec((DBLK, D), lambda i: (i, 0)),
        pl.BlockSpec((1, DBLK, D), lambda i: (0, i, 0)),
        pl.BlockSpec((1, DBLK, D), lambda i: (1, i, 0)),
        pl.BlockSpec((1, DBLK, D), lambda i: (0, i, 0)),
        pl.BlockSpec((1, DBLK, D), lambda i: (1, i, 0)),
        pl.BlockSpec((D, D), full),
        pl.BlockSpec((D, D), full),
        pl.BlockSpec((1, D), full),
        pl.BlockSpec((1, D), full),
        pl.BlockSpec((1, D), full),
    ]
    out_shape = [jax.ShapeDtypeStruct((N, D), jnp.float32)]
    out_specs = [pl.BlockSpec((DBLK, D), lambda i: (i, 0))]
    scratch_shapes = []
    if last:
        out_shape.append(jax.ShapeDtypeStruct((2, D), jnp.float32))
        out_specs.append(pl.BlockSpec((2, D), full))
        scratch_shapes.append(pltpu.VMEM((2, D), jnp.float32))
    else:
        in_specs += [pl.BlockSpec((D, D), full), pl.BlockSpec((D, D), full)]
        out_shape += [jax.ShapeDtypeStruct((N, D), jnp.float32),
                      jax.ShapeDtypeStruct((N, D), jnp.float32)]
        out_specs += [pl.BlockSpec((DBLK, D), lambda i: (i, 0)),
                      pl.BlockSpec((DBLK, D), lambda i: (i, 0))]
    return pl.pallas_call(
        body,
        grid=(NBLK,),
        in_specs=in_specs,
        out_specs=tuple(out_specs),
        out_shape=tuple(out_shape),
        scratch_shapes=scratch_shapes,
    )


def kernel(type_indices, adjacency, table, W_msg, b_msg, W_upd, b_upd,
           gamma, beta):
    f32 = jnp.float32
    src = adjacency[:, 0]
    tgt = adjacency[:, 1]
    npad = EPAD - E
    zpad = jnp.zeros((npad,), jnp.int32)
    gpad = N + (jnp.arange(npad, dtype=jnp.int32) % (R - N))
    srcp = jnp.concatenate([src, zpad])
    tgtg = jnp.concatenate([tgt, zpad])
    tgts = jnp.concatenate([tgt, gpad])
    ti = jnp.concatenate(
        [type_indices, jnp.zeros((NPAD - N,), jnp.int32)])
    zeros_hbm = jnp.zeros((CK, D), f32)
    ones_hbm = jnp.ones((CK, D), f32)
    WsT = [W_msg[l][:, :D].T for l in range(LAYERS)]
    WtT = [W_msg[l][:, D:].T for l in range(LAYERS)]
    Wu1T = [W_upd[l][:, :D].T for l in range(LAYERS)]
    Wu2T = [W_upd[l][:, D:].T for l in range(LAYERS)]
    tabA0 = table @ WsT[0]
    tabB0 = table @ WtT[0]

    x, A, B = _make_sc_init()(table, tabA0, tabB0, ti)
    cnt = _make_count_kernel()(tgts, ones_hbm, zeros_hbm)
    if isinstance(cnt, (tuple, list)):
        cnt = cnt[0]
    for l in range(LAYERS):
        part = _make_edge_kernel()(A, B, srcp, tgtg, tgts, b_msg[l],
                                   zeros_hbm)
        if isinstance(part, (tuple, list)):
            part = part[0]
        last = l == LAYERS - 1
        dense = _make_dense(last, x.shape[0])
        args = [x, part, part, cnt, cnt, Wu1T[l], Wu2T[l],
                b_upd[l].reshape(1, D), gamma[l].reshape(1, D),
                beta[l].reshape(1, D)]
        if last:
            x, pooled = dense(*args)
        else:
            args += [WsT[l + 1], WtT[l + 1]]
            x, A, B = dense(*args)
    return pooled.reshape(2 * D)


# speedup vs baseline: 3.0842x; 1.0015x over previous
import functools

import jax
import jax.numpy as jnp
from jax import lax
from jax.experimental import pallas as pl
from jax.experimental.pallas import tpu as pltpu
from jax.experimental.pallas import tpu_sc as plsc

N = 10000
E = 160000
D = 128
LAYERS = 3
T = 64

NC = 2
NS = 16
NW = NC * NS
VL = 16

CK = 64
ECHUNKS = 80
EW = CK * ECHUNKS
EPAD = EW * NW

R = 10240
RW = R // NS

NCHUNK_INIT = 5
NPAD = NW * NCHUNK_INIT * CK

DBLK = 1000
NBLK = N // DBLK

def _mesh():
    return plsc.VectorSubcoreMesh(core_axis_name="c", subcore_axis_name="s",
                                  num_cores=NC, num_subcores=NS)


@functools.lru_cache(maxsize=None)
def _make_sc_init():
    @functools.partial(
        pl.kernel,
        out_type=(
            jax.ShapeDtypeStruct((NPAD, D), jnp.float32),
            jax.ShapeDtypeStruct((NPAD, D), jnp.float32),
            jax.ShapeDtypeStruct((NPAD, D), jnp.float32),
        ),
        mesh=_mesh(),
        scratch_types=[
            pltpu.VMEM((CK,), jnp.int32),
            pltpu.VMEM((CK, D), jnp.float32),
            pltpu.SemaphoreType.DMA,
        ],
    )
    def _sc_init(tabx, taba, tabb, idx_h, outx, outa, outb, idxv, buf, sem):
        wid = lax.axis_index("s") * NC + lax.axis_index("c")
        for c in range(NCHUNK_INIT):
            base = (wid * NCHUNK_INIT + c) * CK
            pltpu.sync_copy(idx_h.at[pl.ds(base, CK)], idxv)
            for tab, out in ((tabx, outx), (taba, outa), (tabb, outb)):
                pltpu.async_copy(tab.at[idxv], buf, sem).wait()
                pltpu.sync_copy(buf, out.at[pl.ds(base, CK)])

    return _sc_init


@functools.lru_cache(maxsize=None)
def _make_edge_kernel():
    def body(a_hbm, b_hbm, src_h, tgtg_h, tgts_h, bias_hbm, zeros_hbm,
             part, msgacc, gxs0, gxs1, gxt0, gxt1, scx0, scx1,
             rowsa0, rowsa1, rowsb0, rowsb1, biasv,
             gisem0, gisem1, gsem0, gsem1, ssem0, ssem1, smsem0, smsem1):
        cid = lax.axis_index("c")
        sid = lax.axis_index("s")
        wid = sid * NC + cid
        gxs = (gxs0, gxs1)
        gxt = (gxt0, gxt1)
        scx = (scx0, scx1)
        rowsa = (rowsa0, rowsa1)
        rowsb = (rowsb0, rowsb1)
        gisem = (gisem0, gisem1)
        gsem = (gsem0, gsem1)
        ssem = (ssem0, ssem1)
        smsem = (smsem0, smsem1)

        pltpu.sync_copy(zeros_hbm, rowsa0)

        def zloop(k, carry):
            pltpu.sync_copy(rowsa0, msgacc.at[pl.ds(sid * RW + k * CK, CK)])
            return carry

        lax.fori_loop(0, RW // CK, zloop, 0)

        pltpu.sync_copy(bias_hbm, biasv)
        plsc.subcore_barrier()

        bjs = [biasv[pl.ds(j * VL, VL)] for j in range(D // VL)]

        def choff(cc):
            return wid * EW + jnp.minimum(cc, ECHUNKS - 1) * CK

        def gidx(cc, b):
            off = choff(cc)
            return (pltpu.make_async_copy(src_h.at[pl.ds(off, CK)], gxs[b],
                                          gisem[b]),
                    pltpu.make_async_copy(tgtg_h.at[pl.ds(off, CK)], gxt[b],
                                          gisem[b]))

        def gsc(cc, b):
            return pltpu.make_async_copy(tgts_h.at[pl.ds(choff(cc), CK)],
                                         scx[b], ssem[b])

        def ggat(b):
            return (pltpu.make_async_copy(a_hbm.at[gxs[b]], rowsa[b],
                                          gsem[b]),
                    pltpu.make_async_copy(b_hbm.at[gxt[b]], rowsb[b],
                                          gsem[b]))

        def gsct(b):
            return pltpu.make_async_copy(rowsa[b], msgacc.at[scx[b]],
                                         smsem[b])

        for d in gidx(0, 0) + (gsc(0, 0),):
            d.start()
        for d in gidx(0, 0):
            d.wait()
        for d in ggat(0):
            d.start()
        for d in gidx(1, 1) + (gsc(1, 1),):
            d.start()

        @pl.loop(0, ECHUNKS, step=2)
        def _(c0):
            for b in range(2):
                cc = c0 + b
                nb = 1 - b
                for dd in gidx(cc + 1, nb):
                    dd.wait()

                @pl.when(cc > 0)
                def _():
                    gsct(nb).wait()
                    gsc(cc + 1, nb).start()

                for dd in ggat(nb):
                    dd.start()
                for dd in ggat(b):
                    dd.wait()
                for dd in gidx(cc + 2, b):
                    dd.start()
                ra, rb = rowsa[b], rowsb[b]

                @pl.loop(0, CK, unroll=4)
                def _(r):
                    for j in range(D // VL):
                        sl = pl.ds(j * VL, VL)
                        ra[r, sl] = jnp.maximum(
                            ra[r, sl] + rb[r, sl] + bjs[j], 0.0)

                gsc(cc, b).wait()
                pltpu.async_copy(ra, msgacc.at[scx[b]], smsem[b], add=True)

        for d in ggat(0):
            d.wait()
        for d in gidx(ECHUNKS, 1):
            d.wait()
        gsc(ECHUNKS, 0).wait()
        gsct(1).wait()
        plsc.subcore_barrier()

        sl = pl.ds(sid * RW, RW)
        pltpu.sync_copy(msgacc.at[sl], part.at[cid, sl])

    return pl.kernel(
        body,
        out_type=jax.ShapeDtypeStruct((NC, R, D), jnp.float32),
        mesh=_mesh(),
        scratch_types=[
            pltpu.VMEM_SHARED((R, D), jnp.float32),
            pltpu.VMEM((CK,), jnp.int32),
            pltpu.VMEM((CK,), jnp.int32),
            pltpu.VMEM((CK,), jnp.int32),
            pltpu.VMEM((CK,), jnp.int32),
            pltpu.VMEM((CK,), jnp.int32),
            pltpu.VMEM((CK,), jnp.int32),
            pltpu.VMEM((CK, D), jnp.float32),
            pltpu.VMEM((CK, D), jnp.float32),
            pltpu.VMEM((CK, D), jnp.float32),
            pltpu.VMEM((CK, D), jnp.float32),
            pltpu.VMEM((D,), jnp.float32),
            pltpu.SemaphoreType.DMA,
            pltpu.SemaphoreType.DMA,
            pltpu.SemaphoreType.DMA,
            pltpu.SemaphoreType.DMA,
            pltpu.SemaphoreType.DMA,
            pltpu.SemaphoreType.DMA,
            pltpu.SemaphoreType.DMA,
            pltpu.SemaphoreType.DMA,
        ])


@functools.lru_cache(maxsize=None)
def _make_count_kernel():
    def body(tgts_h, ones_hbm, zeros_hbm, cnt, cntacc, scatx, onesv, rowsz):
        cid = lax.axis_index("c")
        sid = lax.axis_index("s")
        wid = sid * NC + cid
        pltpu.sync_copy(zeros_hbm, rowsz)

        def zloop(k, carry):
            pltpu.sync_copy(rowsz, cntacc.at[pl.ds(sid * RW + k * CK, CK)])
            return carry

        lax.fori_loop(0, RW // CK, zloop, 0)
        pltpu.sync_copy(ones_hbm, onesv)
        plsc.subcore_barrier()

        def chunk(c, carry):
            off = wid * EW + c * CK
            pltpu.sync_copy(tgts_h.at[pl.ds(off, CK)], scatx)
            pltpu.sync_copy(onesv, cntacc.at[scatx], add=True)
            return carry

        lax.fori_loop(0, ECHUNKS, chunk, 0)
        plsc.subcore_barrier()
        sl = pl.ds(sid * RW, RW)
        pltpu.sync_copy(cntacc.at[sl], cnt.at[cid, sl])

    return pl.kernel(
        body,
        out_type=jax.ShapeDtypeStruct((NC, R, D), jnp.float32),
        mesh=_mesh(),
        scratch_types=[
            pltpu.VMEM_SHARED((R, D), jnp.float32),
            pltpu.VMEM((CK,), jnp.int32),
            pltpu.VMEM((CK, D), jnp.float32),
            pltpu.VMEM((CK, D), jnp.float32),
        ])


def _make_dense(last, x_rows):
    def body(x_ref, p0, p1, c0, c1, wu1, wu2, bu, gm, bt, *rest):
        if last:
            xn_ref, pool_ref, acc = rest
        else:
            ws, wt = rest[:2]
            xn_ref, a_ref, b_ref = rest[2:]
        x = x_ref[...]
        p = p0[...][0] + p1[...][0]
        c = c0[...][0] + c1[...][0]
        m = p / jnp.maximum(c, 1.0)
        u = jnp.dot(x, wu1[...], preferred_element_type=jnp.float32)
        u += jnp.dot(m, wu2[...], preferred_element_type=jnp.float32)
        u = jnp.maximum(u + bu[...], 0.0)
        h = u + x
        mu = jnp.mean(h, axis=-1, keepdims=True)
        var = jnp.mean((h - mu) ** 2, axis=-1, keepdims=True)
        xn = (h - mu) / jnp.sqrt(var + 1e-5) * gm[...] + bt[...]
        xn_ref[...] = xn
        if last:
            i = pl.program_id(0)
            bs = jnp.sum(xn, axis=0, keepdims=True)
            bm = jnp.max(xn, axis=0, keepdims=True)

            @pl.when(i == 0)
            def _():
                acc[0:1] = bs
                acc[1:2] = bm

            @pl.when(i > 0)
            def _():
                acc[0:1] = acc[0:1] + bs
                acc[1:2] = jnp.maximum(acc[1:2], bm)

            @pl.when(i == NBLK - 1)
            def _():
                pool_ref[0:1] = acc[0:1] * (1.0 / N)
                pool_ref[1:2] = acc[1:2]
        else:
            a_ref[...] = jnp.dot(xn, ws[...],
                                 preferred_element_type=jnp.float32)
            b_ref[...] = jnp.dot(xn, wt[...],
                                 preferred_element_type=jnp.float32)

    full = lambda i: (0, 0)
    in_specs = [
        pl.BlockSpec((DBLK, D), lambda i: (i, 0)),
        pl.BlockSpec((1, DBLK, D), lambda i: (0, i, 0)),
        pl.BlockSpec((1, DBLK, D), lambda i: (1, i, 0)),
        pl.BlockSpec((1, DBLK, D), lambda i: (0, i, 0)),
        pl.BlockSpec((1, DBLK, D), lambda i: (1, i, 0)),
        pl.BlockSpec((D, D), full),
        pl.BlockSpec((D, D), full),
        pl.BlockSpec((1, D), full),
        pl.BlockSpec((1, D), full),
        pl.BlockSpec((1, D), full),
    ]
    out_shape = [jax.ShapeDtypeStruct((N, D), jnp.float32)]
    out_specs = [pl.BlockSpec((DBLK, D), lambda i: (i, 0))]
    scratch_shapes = []
    if last:
        out_shape.append(jax.ShapeDtypeStruct((2, D), jnp.float32))
        out_specs.append(pl.BlockSpec((2, D), full))
        scratch_shapes.append(pltpu.VMEM((2, D), jnp.float32))
    else:
        in_specs += [pl.BlockSpec((D, D), full), pl.BlockSpec((D, D), full)]
        out_shape += [jax.ShapeDtypeStruct((N, D), jnp.float32),
                      jax.ShapeDtypeStruct((N, D), jnp.float32)]
        out_specs += [pl.BlockSpec((DBLK, D), lambda i: (i, 0)),
                      pl.BlockSpec((DBLK, D), lambda i: (i, 0))]
    return pl.pallas_call(
        body,
        grid=(NBLK,),
        in_specs=in_specs,
        out_specs=tuple(out_specs),
        out_shape=tuple(out_shape),
        scratch_shapes=scratch_shapes,
    )


def kernel(type_indices, adjacency, table, W_msg, b_msg, W_upd, b_upd,
           gamma, beta):
    f32 = jnp.float32
    src = adjacency[:, 0]
    tgt = adjacency[:, 1]
    npad = EPAD - E
    zpad = jnp.zeros((npad,), jnp.int32)
    gpad = N + (jnp.arange(npad, dtype=jnp.int32) % (R - N))
    srcp = jnp.concatenate([src, zpad])
    tgtg = jnp.concatenate([tgt, zpad])
    tgts = jnp.concatenate([tgt, gpad])
    ti = jnp.concatenate(
        [type_indices, jnp.zeros((NPAD - N,), jnp.int32)])
    zeros_hbm = jnp.zeros((CK, D), f32)
    ones_hbm = jnp.ones((CK, D), f32)
    WsT = [W_msg[l][:, :D].T for l in range(LAYERS)]
    WtT = [W_msg[l][:, D:].T for l in range(LAYERS)]
    Wu1T = [W_upd[l][:, :D].T for l in range(LAYERS)]
    Wu2T = [W_upd[l][:, D:].T for l in range(LAYERS)]
    tabA0 = table @ WsT[0]
    tabB0 = table @ WtT[0]

    x, A, B = _make_sc_init()(table, tabA0, tabB0, ti)
    cnt = _make_count_kernel()(tgts, ones_hbm, zeros_hbm)
    if isinstance(cnt, (tuple, list)):
        cnt = cnt[0]
    for l in range(LAYERS):
        part = _make_edge_kernel()(A, B, srcp, tgtg, tgts, b_msg[l],
                                   zeros_hbm)
        if isinstance(part, (tuple, list)):
            part = part[0]
        last = l == LAYERS - 1
        dense = _make_dense(last, x.shape[0])
        args = [x, part, part, cnt, cnt, Wu1T[l], Wu2T[l],
                b_upd[l].reshape(1, D), gamma[l].reshape(1, D),
                beta[l].reshape(1, D)]
        if last:
            x, pooled = dense(*args)
        else:
            args += [WsT[l + 1], WtT[l + 1]]
            x, A, B = dense(*args)
    return pooled.reshape(2 * D)
